# baseline (speedup 1.0000x reference)
import functools

import jax
import jax.numpy as jnp
from jax import lax
from jax.experimental import pallas as pl
from jax.experimental.pallas import tpu as pltpu
from jax.experimental.pallas import tpu_sc as plsc


def kernel(indices, primitives):
    B, L = indices.shape
    V, D = primitives.shape
    N = B * L
    flat_idx = indices.reshape(N)
    table_flat = primitives.T.reshape(V * D)

    info = plsc.get_sparse_core_info()
    NC, NS = info.num_cores, info.num_subcores
    NW = NC * NS
    NDG = 8
    NBR = NW // NDG
    b_per_w = B // NBR
    BBLK = 8
    n_blk = b_per_w // BBLK
    DR = D // NDG

    mesh = plsc.VectorSubcoreMesh(core_axis_name="c", subcore_axis_name="s")

    @functools.partial(
        pl.kernel,
        mesh=mesh,
        compiler_params=pltpu.CompilerParams(
            use_tc_tiling_on_sc=True, needs_layout_passes=False),
        out_type=jax.ShapeDtypeStruct((B, D, L), jnp.float32),
        scratch_types=[
            pltpu.VMEM((DR * V,), jnp.float32),
            pltpu.VMEM((BBLK * L,), jnp.int32),
            pltpu.VMEM((2, DR, L), jnp.float32),
            pltpu.SemaphoreType.DMA((2,)),
        ],
    )
    def gather_k(table_hbm, idx_hbm, out_hbm, table_v, idx_v, out_v, osem):
        wid = lax.axis_index("s") * NC + lax.axis_index("c")
        jd = wid % NDG
        b0 = (wid // NDG) * b_per_w

        pltpu.sync_copy(table_hbm.at[pl.ds(jd * DR * V, DR * V)], table_v)

        def wait_out(sb):
            pltpu.make_async_copy(
                out_v.at[sb], out_hbm.at[0, pl.ds(0, DR)], osem.at[sb]).wait()

        def blk_body(g, carry):
            pltpu.sync_copy(
                idx_hbm.at[pl.ds((b0 + g * BBLK) * L, BBLK * L)], idx_v)
            for rb in range(BBLK):
                sb = rb % 2
                if rb >= 2:
                    wait_out(sb)
                else:
                    @pl.when(g > 0)
                    def _():
                        wait_out(sb)

                @plsc.parallel_loop(0, L // 16, unroll=8)
                def j_body(j):
                    idx16 = idx_v[pl.ds(rb * L + j * 16, 16)]
                    for dr in range(DR):
                        vals = plsc.load_gather(
                            table_v.at[pl.ds(dr * V, V)], [idx16])
                        out_v[sb, dr, pl.ds(j * 16, 16)] = vals
                pltpu.async_copy(
                    out_v.at[sb],
                    out_hbm.at[b0 + g * BBLK + rb, pl.ds(jd * DR, DR)],
                    osem.at[sb])
            return carry

        lax.fori_loop(0, n_blk, blk_body, 0)
        wait_out(0)
        wait_out(1)

    out = gather_k(table_flat, flat_idx)
    return jnp.swapaxes(out, 1, 2)


# speedup vs baseline: 1.1570x; 1.1570x over previous
import functools

import jax
import jax.numpy as jnp
from jax import lax
from jax.experimental import pallas as pl
from jax.experimental.pallas import tpu as pltpu
from jax.experimental.pallas import tpu_sc as plsc


def kernel(indices, primitives):
    B, L = indices.shape
    V, D = primitives.shape
    N = B * L
    flat_idx = indices.reshape(N)
    table_flat = primitives.T.reshape(V * D)

    info = plsc.get_sparse_core_info()
    NC, NS = info.num_cores, info.num_subcores
    NW = NC * NS
    NDG = 8
    NBR = NW // NDG
    b_per_w = B // NBR
    BBLK = 8
    n_blk = b_per_w // BBLK
    DR = D // NDG

    mesh = plsc.VectorSubcoreMesh(core_axis_name="c", subcore_axis_name="s")

    @functools.partial(
        pl.kernel,
        mesh=mesh,
        compiler_params=pltpu.CompilerParams(
            use_tc_tiling_on_sc=True, needs_layout_passes=False),
        out_type=jax.ShapeDtypeStruct((B, D, L), jnp.float32),
        scratch_types=[
            pltpu.VMEM((DR * V,), jnp.float32),
            pltpu.VMEM((2, BBLK * L), jnp.int32),
            pltpu.VMEM((2, DR, L), jnp.float32),
            pltpu.SemaphoreType.DMA((2,)),
            pltpu.SemaphoreType.DMA((2,)),
        ],
    )
    def gather_k(table_hbm, idx_hbm, out_hbm, table_v, idx_v, out_v, osem,
                 isem):
        wid = lax.axis_index("s") * NC + lax.axis_index("c")
        jd = wid % NDG
        b0 = (wid // NDG) * b_per_w

        pltpu.sync_copy(table_hbm.at[pl.ds(jd * DR * V, DR * V)], table_v)

        def wait_out(sb):
            pltpu.make_async_copy(
                out_v.at[sb], out_hbm.at[0, pl.ds(0, DR)], osem.at[sb]).wait()

        def fire_idx(g, si):
            pltpu.async_copy(
                idx_hbm.at[pl.ds((b0 + g * BBLK) * L, BBLK * L)],
                idx_v.at[si], isem.at[si])

        def wait_idx(si):
            pltpu.make_async_copy(
                idx_hbm.at[pl.ds(0, BBLK * L)], idx_v.at[si],
                isem.at[si]).wait()

        fire_idx(0, 0)

        def blk_pair_body(g2, carry):
            for p in range(2):
                g = 2 * g2 + p
                wait_idx(p)

                @pl.when(g + 1 < n_blk)
                def _():
                    fire_idx(g + 1, 1 - p)

                for rb in range(BBLK):
                    sb = rb % 2
                    if rb >= 2:
                        wait_out(sb)
                    else:
                        @pl.when(g > 0)
                        def _():
                            wait_out(sb)

                    @plsc.parallel_loop(0, L // 16, unroll=8)
                    def j_body(j):
                        idx16 = idx_v[p, pl.ds(rb * L + j * 16, 16)]
                        for dr in range(DR):
                            vals = plsc.load_gather(
                                table_v.at[pl.ds(dr * V, V)], [idx16])
                            out_v[sb, dr, pl.ds(j * 16, 16)] = vals
                    pltpu.async_copy(
                        out_v.at[sb],
                        out_hbm.at[b0 + g * BBLK + rb, pl.ds(jd * DR, DR)],
                        osem.at[sb])
            return carry

        lax.fori_loop(0, n_blk // 2, blk_pair_body, 0)
        wait_out(0)
        wait_out(1)

    out = gather_k(table_flat, flat_idx)
    return jnp.swapaxes(out, 1, 2)
